# Initial kernel scaffold; baseline (speedup 1.0000x reference)
import jax
import jax.numpy as jnp
from jax.experimental import pallas as pl


def kernel(features_batch, hidden, seq_lens):
    raise NotImplementedError("write your pallas kernel here")


# speedup vs baseline: 2.3653x; 2.3653x over previous
import jax
import jax.numpy as jnp
from jax.experimental import pallas as pl
from jax.experimental.pallas import tpu as pltpu

_B = 8
_L = 512
_F = 256
_NEG_INF = float("-inf")


def _loss_kernel(seq_ref, h_ref, x_ref, out_ref, m_scr, s_scr, d_scr):
    r = pl.program_id(0)
    c = pl.program_id(1)
    d = r // _B
    i = r % _B

    a = h_ref[0]
    x = x_ref[0]

    logits = jax.lax.dot_general(
        a, x, (((1,), (1,)), ((), ())), preferred_element_type=jnp.float32
    )

    t_iota = jax.lax.broadcasted_iota(jnp.int32, (_L, _L), 0)
    s_iota = jax.lax.broadcasted_iota(jnp.int32, (_L, _L), 1)
    l_j = jnp.maximum(seq_ref[c], 1)
    masked = jnp.where(s_iota < l_j, logits, _NEG_INF)
    tile_max = jnp.max(masked, axis=1, keepdims=True)

    @pl.when(c == 0)
    def _init():
        m_scr[...] = jnp.full((_L, 128), _NEG_INF, jnp.float32)
        s_scr[...] = jnp.zeros((_L, 128), jnp.float32)

    m = m_scr[:, 0:1]
    s = s_scr[:, 0:1]
    new_m = jnp.maximum(m, tile_max)
    p_sum = jnp.sum(jnp.exp(masked - new_m), axis=1, keepdims=True)
    s_new = s * jnp.exp(m - new_m) + p_sum
    m_scr[...] = jnp.broadcast_to(new_m, (_L, 128))
    s_scr[...] = jnp.broadcast_to(s_new, (_L, 128))

    @pl.when(c == i)
    def _diag():
        l_i = jnp.maximum(seq_ref[i], 1)
        off = jnp.where(d == 0, 1, -1)
        sel = s_iota == (t_iota + off)
        dsum = jnp.sum(jnp.where(sel, logits, 0.0), axis=1, keepdims=True)
        t_col = jax.lax.broadcasted_iota(jnp.int32, (_L, 1), 0)
        lo = jnp.where(d == 0, 0, 1)
        hi = jnp.where(d == 0, l_i - 1, _L)
        valid = (t_col >= lo) & (t_col < hi)
        d_scr[...] = jnp.broadcast_to(jnp.where(valid, dsum, 0.0), (_L, 128))

    @pl.when(c == _B - 1)
    def _finalize():
        l_i = jnp.maximum(seq_ref[i], 1)
        lse = jnp.logaddexp(new_m + jnp.log(s_new), jnp.log(16.0))
        t_col = jax.lax.broadcasted_iota(jnp.int32, (_L, 1), 0)
        contrib = jnp.where(t_col < l_i, d_scr[:, 0:1] - lse, 0.0)
        val = -jnp.sum(contrib) / (l_i.astype(jnp.float32) * _B)

        @pl.when(r == 0)
        def _zero():
            out_ref[...] = jnp.zeros((8, 128), jnp.float32)

        row_iota = jax.lax.broadcasted_iota(jnp.int32, (8, 128), 0)
        lane_iota = jax.lax.broadcasted_iota(jnp.int32, (8, 128), 1)
        add = jnp.where((row_iota == d) & (lane_iota == 0), val, 0.0)
        out_ref[...] = out_ref[...] + add


def kernel(features_batch, hidden, seq_lens):
    seq_lens = jnp.maximum(seq_lens, 1).astype(jnp.int32)
    grid_spec = pltpu.PrefetchScalarGridSpec(
        num_scalar_prefetch=1,
        grid=(2 * _B, _B),
        in_specs=[
            pl.BlockSpec((1, _L, _F), lambda r, c, seq: (r % _B, 0, r // _B)),
            pl.BlockSpec((1, _L, _F), lambda r, c, seq: (c, 0, 0)),
        ],
        out_specs=pl.BlockSpec((8, 128), lambda r, c, seq: (0, 0)),
        scratch_shapes=[
            pltpu.VMEM((_L, 128), jnp.float32),
            pltpu.VMEM((_L, 128), jnp.float32),
            pltpu.VMEM((_L, 128), jnp.float32),
        ],
    )
    out = pl.pallas_call(
        _loss_kernel,
        grid_spec=grid_spec,
        out_shape=jax.ShapeDtypeStruct((8, 128), jnp.float32),
    )(seq_lens, hidden, features_batch)
    return (out[0, 0:1], out[1, 0:1])
